# baseline (speedup 1.0000x reference)
import functools

import jax
import jax.numpy as jnp
from jax import lax
from jax.experimental import pallas as pl
from jax.experimental.pallas import tpu as pltpu

NS = 16384
NQ = 4096
D = 1024
C = 64
K = 16
ST = 16
QT = 8
SB = NS // ST
QB = NQ // QT

_HI = jax.lax.Precision.HIGHEST


def _protos_body(feat_ref, lab_ref, psum_ref, cnt_ref):
    i = pl.program_id(0)
    labels = lab_ref[0, 0, :]
    oh = (lax.broadcasted_iota(jnp.int32, (C, SB), 0) == labels[None, :]
          ).astype(jnp.float32)
    part = lax.dot_general(oh, feat_ref[...], (((1,), (0,)), ((), ())),
                           precision=_HI, preferred_element_type=jnp.float32)
    cnt_part = jnp.sum(oh, axis=1)[None, :]

    @pl.when(i == 0)
    def _():
        psum_ref[...] = part
        cnt_ref[...] = cnt_part

    @pl.when(i > 0)
    def _():
        psum_ref[...] = psum_ref[...] + part
        cnt_ref[...] = cnt_ref[...] + cnt_part

    @pl.when(i == ST - 1)
    def _():
        psum_ref[...] = psum_ref[...] / jnp.maximum(cnt_ref[0, :], 1.0)[:, None]


def _extract_topk(vals, keys, nk):
    z = vals
    kk = keys
    d_cols = []
    k_cols = []
    for _ in range(nk):
        m = jnp.min(z, axis=1, keepdims=True)
        key_cand = jnp.where(z == m, kk, jnp.inf)
        km = jnp.min(key_cand, axis=1, keepdims=True)
        z = jnp.where(key_cand == km, jnp.inf, z)
        d_cols.append(m)
        k_cols.append(km)
    return jnp.concatenate(d_cols, axis=1), jnp.concatenate(k_cols, axis=1)


def _main_body(q_ref, s_ref, slab_ref, qlab_ref, protos_ref,
               score_ref, scores_ref, idx_ref, kd_ref, kpred_ref,
               loss_ref, acc_ref,
               run_d, run_k, ce_s, knn_s, accn_s):
    qt = pl.program_id(0)
    st = pl.program_id(1)

    q = q_ref[...]
    s = s_ref[...]
    qq = jnp.sum(q * q, axis=1, keepdims=True)
    ss = jnp.sum(s * s, axis=1)
    qs = lax.dot_general(q, s, (((1,), (1,)), ((), ())),
                         preferred_element_type=jnp.float32)
    d2 = (qq + ss[None, :]) - 2.0 * qs

    labels_s = slab_ref[0, 0, :]
    col = st * SB + lax.broadcasted_iota(jnp.int32, (QB, SB), 1)
    keyf = (col * 64 + labels_s[None, :]).astype(jnp.float32)

    tile_d, tile_k = _extract_topk(d2, keyf, K)

    @pl.when(st == 0)
    def _():
        run_d[...] = jnp.full((QB, K), jnp.inf, jnp.float32)
        run_k[...] = jnp.zeros((QB, K), jnp.float32)

    cat_d = jnp.concatenate([run_d[...], tile_d], axis=1)
    cat_k = jnp.concatenate([run_k[...], tile_k], axis=1)
    new_d, new_k = _extract_topk(cat_d, cat_k, K)
    run_d[...] = new_d
    run_k[...] = new_k

    @pl.when(st == ST - 1)
    def _():
        kd_ref[...] = new_d
        idx_f = jnp.floor(new_k * (1.0 / 64.0))
        lab_f = new_k - 64.0 * idx_f
        idx_ref[...] = idx_f.astype(jnp.int32)
        knn_labels = lab_f.astype(jnp.int32)

        ciota = lax.broadcasted_iota(jnp.int32, (QB, C), 1)
        votes = jnp.zeros((QB, C), jnp.float32)
        for j in range(K):
            votes = votes + (knn_labels[:, j:j + 1] == ciota).astype(jnp.float32)
        vm = jnp.max(votes, axis=1, keepdims=True)
        kpred = jnp.min(jnp.where(votes == vm, ciota, C), axis=1)
        kpred_ref[...] = kpred[:, None]

        qlab = qlab_ref[0, 0, :]
        knn_err = jnp.sum((knn_labels != qlab[:, None]).astype(jnp.float32))

        protos = protos_ref[...]
        pp = jnp.sum(protos * protos, axis=1)
        pdot = lax.dot_general(q, protos, (((1,), (1,)), ((), ())),
                               preferred_element_type=jnp.float32)
        scoreb = -((qq + pp[None, :]) - 2.0 * pdot)
        mrow = jnp.max(scoreb, axis=1, keepdims=True)
        shifted = scoreb - mrow
        e = jnp.exp(shifted)
        sume = jnp.sum(e, axis=1, keepdims=True)
        score_ref[...] = scoreb
        scores_ref[...] = e / sume
        logp = shifted - jnp.log(sume)

        ce_hit = jnp.sum(jnp.where(ciota == qlab[:, None], logp, 0.0))
        ppred = jnp.min(jnp.where(scoreb == mrow, ciota, C), axis=1)
        acc_hit = jnp.sum((ppred == qlab).astype(jnp.float32))

        ce_s[0, 0] = jnp.where(qt == 0, 0.0, ce_s[0, 0]) + (-ce_hit)
        knn_s[0, 0] = jnp.where(qt == 0, 0.0, knn_s[0, 0]) + knn_err
        accn_s[0, 0] = jnp.where(qt == 0, 0.0, accn_s[0, 0]) + acc_hit

        @pl.when(qt == QT - 1)
        def _():
            ce = ce_s[0, 0] / float(NQ)
            knn_loss = knn_s[0, 0] / (float(NQ) * float(K))
            loss_ref[...] = jnp.full((1, 1), ce + knn_loss, jnp.float32)
            acc_ref[...] = jnp.full((1, 1), accn_s[0, 0] / float(NQ) * 100.0,
                                    jnp.float32)


@jax.jit
def _run(support_features, support_labels, query_features, query_labels):
    slab3 = support_labels.astype(jnp.int32).reshape(ST, 1, SB)
    qlab3 = query_labels.astype(jnp.int32).reshape(QT, 1, QB)

    protos, _counts = pl.pallas_call(
        _protos_body,
        grid=(ST,),
        in_specs=[
            pl.BlockSpec((SB, D), lambda i: (i, 0)),
            pl.BlockSpec((1, 1, SB), lambda i: (i, 0, 0)),
        ],
        out_specs=[
            pl.BlockSpec((C, D), lambda i: (0, 0)),
            pl.BlockSpec((1, C), lambda i: (0, 0)),
        ],
        out_shape=[
            jax.ShapeDtypeStruct((C, D), jnp.float32),
            jax.ShapeDtypeStruct((1, C), jnp.float32),
        ],
    )(support_features, slab3)

    outs = pl.pallas_call(
        _main_body,
        grid=(QT, ST),
        in_specs=[
            pl.BlockSpec((QB, D), lambda qt, st: (qt, 0)),
            pl.BlockSpec((SB, D), lambda qt, st: (st, 0)),
            pl.BlockSpec((1, 1, SB), lambda qt, st: (st, 0, 0)),
            pl.BlockSpec((1, 1, QB), lambda qt, st: (qt, 0, 0)),
            pl.BlockSpec((C, D), lambda qt, st: (0, 0)),
        ],
        out_specs=[
            pl.BlockSpec((QB, C), lambda qt, st: (qt, 0)),
            pl.BlockSpec((QB, C), lambda qt, st: (qt, 0)),
            pl.BlockSpec((QB, K), lambda qt, st: (qt, 0)),
            pl.BlockSpec((QB, K), lambda qt, st: (qt, 0)),
            pl.BlockSpec((QB, 1), lambda qt, st: (qt, 0)),
            pl.BlockSpec((1, 1), lambda qt, st: (0, 0)),
            pl.BlockSpec((1, 1), lambda qt, st: (0, 0)),
        ],
        out_shape=[
            jax.ShapeDtypeStruct((NQ, C), jnp.float32),
            jax.ShapeDtypeStruct((NQ, C), jnp.float32),
            jax.ShapeDtypeStruct((NQ, K), jnp.int32),
            jax.ShapeDtypeStruct((NQ, K), jnp.float32),
            jax.ShapeDtypeStruct((NQ, 1), jnp.int32),
            jax.ShapeDtypeStruct((1, 1), jnp.float32),
            jax.ShapeDtypeStruct((1, 1), jnp.float32),
        ],
        scratch_shapes=[
            pltpu.VMEM((QB, K), jnp.float32),
            pltpu.VMEM((QB, K), jnp.float32),
            pltpu.SMEM((1, 1), jnp.float32),
            pltpu.SMEM((1, 1), jnp.float32),
            pltpu.SMEM((1, 1), jnp.float32),
        ],
    )(query_features, support_features, slab3, qlab3, protos)

    score, scores, indices, kd, kpred, loss, acc = outs
    return (loss[0, 0], acc[0, 0], score, indices, kd,
            kpred.reshape(NQ), scores)


def kernel(support_features, support_labels, query_features, query_labels, k):
    del k
    return _run(support_features, support_labels, query_features, query_labels)


# speedup vs baseline: 1.1652x; 1.1652x over previous
import functools

import jax
import jax.numpy as jnp
from jax import lax
from jax.experimental import pallas as pl
from jax.experimental.pallas import tpu as pltpu

NS = 16384
NQ = 4096
D = 1024
C = 64
K = 16
ST = 8
QT = 8
SB = NS // ST
QB = NQ // QT

_HI = jax.lax.Precision.HIGHEST


def _protos_body(feat_ref, lab_ref, psum_ref, cnt_ref):
    i = pl.program_id(0)
    labels = lab_ref[0, 0, :]
    oh = (lax.broadcasted_iota(jnp.int32, (C, SB), 0) == labels[None, :]
          ).astype(jnp.float32)
    part = lax.dot_general(oh, feat_ref[...], (((1,), (0,)), ((), ())),
                           precision=_HI, preferred_element_type=jnp.float32)
    cnt_part = jnp.sum(oh, axis=1)[None, :]

    @pl.when(i == 0)
    def _():
        psum_ref[...] = part
        cnt_ref[...] = cnt_part

    @pl.when(i > 0)
    def _():
        psum_ref[...] = psum_ref[...] + part
        cnt_ref[...] = cnt_ref[...] + cnt_part

    @pl.when(i == ST - 1)
    def _():
        psum_ref[...] = psum_ref[...] / jnp.maximum(cnt_ref[0, :], 1.0)[:, None]


def _extract_topk(vals, keys, nk):
    z = vals
    kk = keys
    d_cols = []
    k_cols = []
    for _ in range(nk):
        m = jnp.min(z, axis=1, keepdims=True)
        key_cand = jnp.where(z == m, kk, jnp.inf)
        km = jnp.min(key_cand, axis=1, keepdims=True)
        z = jnp.where(key_cand == km, jnp.inf, z)
        d_cols.append(m)
        k_cols.append(km)
    return jnp.concatenate(d_cols, axis=1), jnp.concatenate(k_cols, axis=1)


def _main_body(q_ref, s_ref, slab_ref, qlab_ref, protos_ref,
               score_ref, scores_ref, idx_ref, kd_ref, kpred_ref,
               loss_ref, acc_ref,
               run_d, run_k, ce_s, knn_s, accn_s):
    qt = pl.program_id(0)
    st = pl.program_id(1)

    q = q_ref[...]
    s = s_ref[...]
    qq = jnp.sum(q * q, axis=1, keepdims=True)
    ss = jnp.sum(s * s, axis=1)
    def _dot1024(sh):
        parts = [lax.dot_general(q[:, j * 256:(j + 1) * 256],
                                 sh[:, j * 256:(j + 1) * 256],
                                 (((1,), (1,)), ((), ())),
                                 preferred_element_type=jnp.float32)
                 for j in range(4)]
        return ((parts[0] + parts[1]) + parts[2]) + parts[3]
    qs = jnp.concatenate(
        [_dot1024(s[h * 1024:(h + 1) * 1024, :])
         for h in range(SB // 1024)], axis=1)
    d2 = (qq + ss[None, :]) - 2.0 * qs

    labels_s = slab_ref[0, 0, :]
    col = st * SB + lax.broadcasted_iota(jnp.int32, (QB, SB), 1)
    keyf = (col * 64 + labels_s[None, :]).astype(jnp.float32)

    tile_d, tile_k = _extract_topk(d2, keyf, K)

    @pl.when(st == 0)
    def _():
        run_d[...] = jnp.full((QB, K), jnp.inf, jnp.float32)
        run_k[...] = jnp.zeros((QB, K), jnp.float32)

    cat_d = jnp.concatenate([run_d[...], tile_d], axis=1)
    cat_k = jnp.concatenate([run_k[...], tile_k], axis=1)
    new_d, new_k = _extract_topk(cat_d, cat_k, K)
    run_d[...] = new_d
    run_k[...] = new_k

    @pl.when(st == ST - 1)
    def _():
        kd_ref[...] = new_d
        idx_f = jnp.floor(new_k * (1.0 / 64.0))
        lab_f = new_k - 64.0 * idx_f
        idx_ref[...] = idx_f.astype(jnp.int32)
        knn_labels = lab_f.astype(jnp.int32)

        ciota = lax.broadcasted_iota(jnp.int32, (QB, C), 1)
        votes = jnp.zeros((QB, C), jnp.float32)
        for j in range(K):
            votes = votes + (knn_labels[:, j:j + 1] == ciota).astype(jnp.float32)
        vm = jnp.max(votes, axis=1, keepdims=True)
        kpred = jnp.min(jnp.where(votes == vm, ciota, C), axis=1)
        kpred_ref[...] = kpred[:, None]

        qlab = qlab_ref[0, 0, :]
        knn_err = jnp.sum((knn_labels != qlab[:, None]).astype(jnp.float32))

        protos = protos_ref[...]
        pp = jnp.sum(protos * protos, axis=1)
        pdot = lax.dot_general(q, protos, (((1,), (1,)), ((), ())),
                               preferred_element_type=jnp.float32)
        scoreb = -((qq + pp[None, :]) - 2.0 * pdot)
        mrow = jnp.max(scoreb, axis=1, keepdims=True)
        shifted = scoreb - mrow
        e = jnp.exp(shifted)
        sume = jnp.sum(e, axis=1, keepdims=True)
        score_ref[...] = scoreb
        scores_ref[...] = e / sume
        logp = shifted - jnp.log(sume)

        ce_hit = jnp.sum(jnp.where(ciota == qlab[:, None], logp, 0.0))
        ppred = jnp.min(jnp.where(scoreb == mrow, ciota, C), axis=1)
        acc_hit = jnp.sum((ppred == qlab).astype(jnp.float32))

        ce_s[0, 0] = jnp.where(qt == 0, 0.0, ce_s[0, 0]) + (-ce_hit)
        knn_s[0, 0] = jnp.where(qt == 0, 0.0, knn_s[0, 0]) + knn_err
        accn_s[0, 0] = jnp.where(qt == 0, 0.0, accn_s[0, 0]) + acc_hit

        @pl.when(qt == QT - 1)
        def _():
            ce = ce_s[0, 0] / float(NQ)
            knn_loss = knn_s[0, 0] / (float(NQ) * float(K))
            loss_ref[...] = jnp.full((1, 1), ce + knn_loss, jnp.float32)
            acc_ref[...] = jnp.full((1, 1), accn_s[0, 0] / float(NQ) * 100.0,
                                    jnp.float32)


@jax.jit
def _run(support_features, support_labels, query_features, query_labels):
    slab3 = support_labels.astype(jnp.int32).reshape(ST, 1, SB)
    qlab3 = query_labels.astype(jnp.int32).reshape(QT, 1, QB)

    protos, _counts = pl.pallas_call(
        _protos_body,
        grid=(ST,),
        in_specs=[
            pl.BlockSpec((SB, D), lambda i: (i, 0)),
            pl.BlockSpec((1, 1, SB), lambda i: (i, 0, 0)),
        ],
        out_specs=[
            pl.BlockSpec((C, D), lambda i: (0, 0)),
            pl.BlockSpec((1, C), lambda i: (0, 0)),
        ],
        out_shape=[
            jax.ShapeDtypeStruct((C, D), jnp.float32),
            jax.ShapeDtypeStruct((1, C), jnp.float32),
        ],
    )(support_features, slab3)

    outs = pl.pallas_call(
        _main_body,
        grid=(QT, ST),
        in_specs=[
            pl.BlockSpec((QB, D), lambda qt, st: (qt, 0)),
            pl.BlockSpec((SB, D), lambda qt, st: (st, 0)),
            pl.BlockSpec((1, 1, SB), lambda qt, st: (st, 0, 0)),
            pl.BlockSpec((1, 1, QB), lambda qt, st: (qt, 0, 0)),
            pl.BlockSpec((C, D), lambda qt, st: (0, 0)),
        ],
        out_specs=[
            pl.BlockSpec((QB, C), lambda qt, st: (qt, 0)),
            pl.BlockSpec((QB, C), lambda qt, st: (qt, 0)),
            pl.BlockSpec((QB, K), lambda qt, st: (qt, 0)),
            pl.BlockSpec((QB, K), lambda qt, st: (qt, 0)),
            pl.BlockSpec((QB, 1), lambda qt, st: (qt, 0)),
            pl.BlockSpec((1, 1), lambda qt, st: (0, 0)),
            pl.BlockSpec((1, 1), lambda qt, st: (0, 0)),
        ],
        out_shape=[
            jax.ShapeDtypeStruct((NQ, C), jnp.float32),
            jax.ShapeDtypeStruct((NQ, C), jnp.float32),
            jax.ShapeDtypeStruct((NQ, K), jnp.int32),
            jax.ShapeDtypeStruct((NQ, K), jnp.float32),
            jax.ShapeDtypeStruct((NQ, 1), jnp.int32),
            jax.ShapeDtypeStruct((1, 1), jnp.float32),
            jax.ShapeDtypeStruct((1, 1), jnp.float32),
        ],
        scratch_shapes=[
            pltpu.VMEM((QB, K), jnp.float32),
            pltpu.VMEM((QB, K), jnp.float32),
            pltpu.SMEM((1, 1), jnp.float32),
            pltpu.SMEM((1, 1), jnp.float32),
            pltpu.SMEM((1, 1), jnp.float32),
        ],
    )(query_features, support_features, slab3, qlab3, protos)

    score, scores, indices, kd, kpred, loss, acc = outs
    return (loss[0, 0], acc[0, 0], score, indices, kd,
            kpred.reshape(NQ), scores)


def kernel(support_features, support_labels, query_features, query_labels, k):
    del k
    return _run(support_features, support_labels, query_features, query_labels)
